# Initial kernel scaffold; baseline (speedup 1.0000x reference)
import jax
import jax.numpy as jnp
from jax.experimental import pallas as pl


def kernel(x_tokens, edge_index, edge_attr_tokens, added_sym_edge, batch, emb, e_w1, e_b1, e_w2, e_b2, n1_w1, n1_b1, n1_w2, n1_b2, n2_w1, n2_b1, n2_w2, n2_b2, gamma, beta):
    raise NotImplementedError("write your pallas kernel here")


# speedup vs baseline: 1.0402x; 1.0402x over previous
import functools

import jax
import jax.numpy as jnp
from jax import lax
from jax.experimental import pallas as pl
from jax.experimental.pallas import tpu as pltpu

N = 10000
E = 160000
D = 128
VOCAB = 3000
TOK = 3
NG = 128
NSYM = 80000

BE = 2000
BN = 2000


def _edge_block_kernel(xr_ref, xc_ref, ea_ref,
                       ew1_ref, eb1_ref, ew2_ref, eb2_ref,
                       nw1_ref, nb1_ref, nw2_ref, nb2_ref,
                       eout_ref, msg_ref):
    xr = xr_ref[...]
    xc = xc_ref[...]
    ea = ea_ref[...]
    w_r = ew1_ref[0:D, :]
    w_c = ew1_ref[D:2 * D, :]
    w_a = ew1_ref[2 * D:3 * D, :]
    h = jnp.dot(xr, w_r, preferred_element_type=jnp.float32)
    h += jnp.dot(xc, w_c, preferred_element_type=jnp.float32)
    h += jnp.dot(ea, w_a, preferred_element_type=jnp.float32)
    h = jnp.maximum(h + eb1_ref[...], 0.0)
    eout = jnp.dot(h, ew2_ref[...], preferred_element_type=jnp.float32) + eb2_ref[...]
    eout_ref[...] = eout

    m_r = nw1_ref[0:D, :]
    m_e = nw1_ref[D:2 * D, :]
    m = jnp.dot(xr, m_r, preferred_element_type=jnp.float32)
    m += jnp.dot(eout, m_e, preferred_element_type=jnp.float32)
    m = jnp.maximum(m + nb1_ref[...], 0.0)
    msg_ref[...] = jnp.dot(m, nw2_ref[...], preferred_element_type=jnp.float32) + nb2_ref[...]


def _edge_mlps(xr, xc, ea, e_w1, e_b1, e_w2, e_b2, n1_w1, n1_b1, n1_w2, n1_b2):
    nblk = E // BE
    full = lambda shape: pl.BlockSpec(shape, lambda i: (0,) * len(shape))
    blk = pl.BlockSpec((BE, D), lambda i: (i, 0))
    return pl.pallas_call(
        _edge_block_kernel,
        grid=(nblk,),
        in_specs=[blk, blk, blk,
                  full((3 * D, D)), full((1, D)), full((D, D)), full((1, D)),
                  full((2 * D, D)), full((1, D)), full((D, D)), full((1, D))],
        out_specs=[blk, blk],
        out_shape=[jax.ShapeDtypeStruct((E, D), jnp.float32),
                   jax.ShapeDtypeStruct((E, D), jnp.float32)],
    )(xr, xc, ea, e_w1, e_b1, e_w2, e_b2, n1_w1, n1_b1, n1_w2, n1_b2)


def _node_block_kernel(xs_ref, agg_ref, batch_row_ref,
                       w1_ref, b1_ref, w2_ref, b2_ref,
                       xout_ref, stats_ref):
    i = pl.program_id(0)
    xs = xs_ref[...]
    agg = agg_ref[...]
    w_x = w1_ref[0:D, :]
    w_a = w1_ref[D:2 * D, :]
    h = jnp.dot(xs, w_x, preferred_element_type=jnp.float32)
    h += jnp.dot(agg, w_a, preferred_element_type=jnp.float32)
    h = jnp.maximum(h + b1_ref[...], 0.0)
    xo = jnp.dot(h, w2_ref[...], preferred_element_type=jnp.float32) + b2_ref[...]
    xout_ref[...] = xo

    gids = lax.broadcasted_iota(jnp.int32, (NG, BN), 0)
    onehot_t = (gids == batch_row_ref[0]).astype(jnp.float32)
    cat = jnp.concatenate([xo, xo * xo, jnp.ones((BN, D), jnp.float32)], axis=1)
    part = jnp.dot(onehot_t, cat, preferred_element_type=jnp.float32)

    @pl.when(i == 0)
    def _():
        stats_ref[...] = jnp.zeros_like(stats_ref)

    stats_ref[...] += part


def _node_mlp_stats(x_sum, agg, batch_row, n2_w1, n2_b1, n2_w2, n2_b2):
    nblk = N // BN
    full = lambda shape: pl.BlockSpec(shape, lambda i: (0,) * len(shape))
    blk = pl.BlockSpec((BN, D), lambda i: (i, 0))
    return pl.pallas_call(
        _node_block_kernel,
        grid=(nblk,),
        in_specs=[blk, blk, pl.BlockSpec((1, 1, BN), lambda i: (i, 0, 0)),
                  full((2 * D, D)), full((1, D)), full((D, D)), full((1, D))],
        out_specs=[blk, full((NG, 3 * D))],
        out_shape=[jax.ShapeDtypeStruct((N, D), jnp.float32),
                   jax.ShapeDtypeStruct((NG, 3 * D), jnp.float32)],
    )(x_sum, agg, batch_row, n2_w1, n2_b1, n2_w2, n2_b2)


def _norm_block_kernel(xo_ref, batch_col_ref, stats_ref, gamma_ref, beta_ref, out_ref):
    xo = xo_ref[...]
    gids = lax.broadcasted_iota(jnp.int32, (BN, NG), 1)
    onehot = (gids == batch_col_ref[...]).astype(jnp.float32)
    st = jnp.dot(onehot, stats_ref[...], preferred_element_type=jnp.float32)
    cnt = jnp.maximum(st[:, 2 * D:2 * D + 1], 1.0)
    mean = st[:, 0:D] / cnt
    var = st[:, D:2 * D] / cnt - mean * mean
    inv = lax.rsqrt(jnp.maximum(var, 0.0) + 1e-5)
    out_ref[...] = (xo - mean) * inv * gamma_ref[...] + beta_ref[...]


def _graph_norm(x_out, batch_col, stats, gamma, beta):
    nblk = N // BN
    full = lambda shape: pl.BlockSpec(shape, lambda i: (0,) * len(shape))
    blk = pl.BlockSpec((BN, D), lambda i: (i, 0))
    return pl.pallas_call(
        _norm_block_kernel,
        grid=(nblk,),
        in_specs=[blk, pl.BlockSpec((BN, 1), lambda i: (i, 0)),
                  full((NG, 3 * D)), full((1, D)), full((1, D))],
        out_specs=blk,
        out_shape=jax.ShapeDtypeStruct((N, D), jnp.float32),
    )(x_out, batch_col, stats, gamma, beta)


def kernel(x_tokens, edge_index, edge_attr_tokens, added_sym_edge, batch, emb,
           e_w1, e_b1, e_w2, e_b2, n1_w1, n1_b1, n1_w2, n1_b2,
           n2_w1, n2_b1, n2_w2, n2_b2, gamma, beta):
    x_tokens = x_tokens.astype(jnp.int32)
    edge_attr_tokens = edge_attr_tokens.astype(jnp.int32)
    edge_index = edge_index.astype(jnp.int32)
    added_sym_edge = added_sym_edge.astype(jnp.int32)
    batch = batch.astype(jnp.int32)

    x_sum = jnp.take(emb, x_tokens, axis=0).sum(axis=-2)
    sym_mask = jnp.zeros((E,), jnp.bool_).at[added_sym_edge].set(True)
    ea_sum = jnp.take(emb, edge_attr_tokens, axis=0).sum(axis=-2)
    ea_sum = jnp.where(sym_mask[:, None], -ea_sum, ea_sum)
    row = edge_index[0]
    col = edge_index[1]
    xr = jnp.take(x_sum, row, axis=0)
    xc = jnp.take(x_sum, col, axis=0)

    r2 = lambda v: v.reshape(1, D)
    edge_out, msg = _edge_mlps(xr, xc, ea_sum, e_w1, r2(e_b1), e_w2, r2(e_b2),
                               n1_w1, r2(n1_b1), n1_w2, r2(n1_b2))

    s = jax.ops.segment_sum(msg, col, num_segments=N)
    cnt = jax.ops.segment_sum(jnp.ones((E, 1), jnp.float32), col, num_segments=N)
    agg = s / jnp.maximum(cnt, 1.0)

    batch_row = batch.reshape(N // BN, 1, BN)
    x_out, stats = _node_mlp_stats(x_sum, agg, batch_row,
                                   n2_w1, r2(n2_b1), n2_w2, r2(n2_b2))

    x_norm = _graph_norm(x_out, batch.reshape(N, 1), stats, r2(gamma), r2(beta))
    return (x_norm, edge_out)


# speedup vs baseline: 5.0367x; 4.8419x over previous
import functools

import jax
import jax.numpy as jnp
from jax import lax
from jax.experimental import pallas as pl
from jax.experimental.pallas import tpu as pltpu
from jax.experimental.pallas import tpu_sc as plsc

N = 10000
E = 160000
D = 128
VOCAB = 3000
TOK = 3
NG = 128
NSYM = 80000

BE = 2000
BN = 2000

NW = 32
CH = 64
ECH = E // CH
CPT = ECH // NW
XTRA = ECH - CPT * NW
EBUF = (CPT + 1) * CH
ZR = 156
PH = 5000
PACC = 5008

def _mesh():
    return plsc.VectorSubcoreMesh(core_axis_name="c", subcore_axis_name="s")


def _wid():
    return lax.axis_index("s") * 2 + lax.axis_index("c")


def _chunk_start(w):
    return w * CPT + jnp.minimum(w, XTRA)


def _vec_sum3(rows3, out, nrows):
    def body(r, _):
        for k in range(D // 16):
            s = pl.ds(k * 16, 16)
            out[r, s] = rows3[0, r, s] + rows3[1, r, s] + rows3[2, r, s]
        return 0
    lax.fori_loop(0, nrows, body, 0)


def _xsum_sc_kernel(emb_hbm, xtok_hbm, out_hbm, t0, t1, t2, rows3, ob, sem):
    w = _wid()
    nch = (N + CH - 1) // CH

    def chunk(j, _):
        t = w + j * NW

        @pl.when(t < nch)
        def _():
            base = jnp.minimum(t * CH, N - CH)
            pltpu.sync_copy(xtok_hbm.at[pl.ds(base, CH)], t0)
            pltpu.sync_copy(xtok_hbm.at[pl.ds(N + base, CH)], t1)
            pltpu.sync_copy(xtok_hbm.at[pl.ds(2 * N + base, CH)], t2)
            c0 = pltpu.async_copy(emb_hbm.at[t0], rows3.at[0], sem)
            c1 = pltpu.async_copy(emb_hbm.at[t1], rows3.at[1], sem)
            c2 = pltpu.async_copy(emb_hbm.at[t2], rows3.at[2], sem)
            c0.wait(); c1.wait(); c2.wait()
            _vec_sum3(rows3, ob, CH)
            pltpu.sync_copy(ob, out_hbm.at[pl.ds(base, CH)])
        return 0

    lax.fori_loop(0, (nch + NW - 1) // NW, chunk, 0)


def _xsum_sc(emb, xtok_t):
    k = pl.kernel(
        _xsum_sc_kernel, mesh=_mesh(),
        compiler_params=pltpu.CompilerParams(needs_layout_passes=False),
        out_type=jax.ShapeDtypeStruct((N, D), jnp.float32),
        scratch_types=[
            pltpu.VMEM((CH,), jnp.int32), pltpu.VMEM((CH,), jnp.int32),
            pltpu.VMEM((CH,), jnp.int32),
            pltpu.VMEM((3, CH, D), jnp.float32),
            pltpu.VMEM((CH, D), jnp.float32),
            pltpu.SemaphoreType.DMA,
        ])
    return k(emb, xtok_t)


SYB = 1600


def _edge_sc_kernel(xsum_hbm, emb2_hbm, eat_hbm, ei_hbm, sym_hbm,
                    xr_hbm, xc_hbm, ea_hbm,
                    mask_v, symb, rowf, colf, tk0, tk1, tk2,
                    ixr, ixc, it0, it1, it2, rows3, xrb, xcb, eab, sem):
    w = _wid()
    c_lo = _chunk_start(w)
    c_hi = _chunk_start(w + 1)
    lo = c_lo * CH
    n_loc = (c_hi - c_lo) * CH

    for hbm, off, buf in ((ei_hbm, 0, rowf), (ei_hbm, E, colf),
                          (eat_hbm, 0, tk0), (eat_hbm, E, tk1),
                          (eat_hbm, 2 * E, tk2)):
        pltpu.sync_copy(hbm.at[pl.ds(off + lo, CPT * CH)],
                        buf.at[pl.ds(0, CPT * CH)])

        @pl.when(w < XTRA)
        def _():
            pltpu.sync_copy(hbm.at[pl.ds(off + lo + CPT * CH, CH)],
                            buf.at[pl.ds(CPT * CH, CH)])

    zero16 = jnp.zeros((16,), jnp.int32)
    one16 = jnp.full((16,), 1, jnp.int32)

    def zeroit(q, _):
        mask_v[pl.ds(q * 16, 16)] = zero16
        return 0
    lax.fori_loop(0, EBUF // 16, zeroit, 0)

    def symscan(s0, _):
        pltpu.sync_copy(sym_hbm.at[pl.ds(s0 * SYB, SYB)], symb)

        def inner(q, _):
            v = symb[pl.ds(q * 16, 16)]
            loc = v - lo
            m = (loc >= 0) & (loc < n_loc)
            locc = jnp.clip(loc, 0, EBUF - 1)
            plsc.store_scatter(mask_v, [locc], one16, mask=m)
            return 0
        lax.fori_loop(0, SYB // 16, inner, 0)
        return 0
    lax.fori_loop(0, NSYM // SYB, symscan, 0)

    def chunk(j, _):
        lb = j * CH
        g = lo + lb
        for q in range(CH // 16):
            sl = pl.ds(lb + q * 16, 16)
            so = pl.ds(q * 16, 16)
            mv = mask_v[sl] * VOCAB
            it0[so] = tk0[sl] + mv
            it1[so] = tk1[sl] + mv
            it2[so] = tk2[sl] + mv
            ixr[so] = rowf[sl]
            ixc[so] = colf[sl]
        cr = pltpu.async_copy(xsum_hbm.at[ixr], xrb, sem)
        cc = pltpu.async_copy(xsum_hbm.at[ixc], xcb, sem)
        c0 = pltpu.async_copy(emb2_hbm.at[it0], rows3.at[0], sem)
        c1 = pltpu.async_copy(emb2_hbm.at[it1], rows3.at[1], sem)
        c2 = pltpu.async_copy(emb2_hbm.at[it2], rows3.at[2], sem)
        cr.wait(); cc.wait(); c0.wait(); c1.wait(); c2.wait()
        _vec_sum3(rows3, eab, CH)
        pltpu.sync_copy(xrb, xr_hbm.at[pl.ds(g, CH)])
        pltpu.sync_copy(xcb, xc_hbm.at[pl.ds(g, CH)])
        pltpu.sync_copy(eab, ea_hbm.at[pl.ds(g, CH)])
        return 0

    lax.fori_loop(0, CPT, chunk, 0)

    @pl.when(w < XTRA)
    def _():
        chunk(CPT, 0)


def _edge_sc(x_sum, emb2, eat_t, edge_index, added_sym_edge):
    vi = lambda shape: pltpu.VMEM(shape, jnp.int32)
    vf = lambda shape: pltpu.VMEM(shape, jnp.float32)
    k = pl.kernel(
        _edge_sc_kernel, mesh=_mesh(),
        compiler_params=pltpu.CompilerParams(needs_layout_passes=False),
        out_type=[jax.ShapeDtypeStruct((E, D), jnp.float32)] * 3,
        scratch_types=[
            vi((EBUF,)), vi((SYB,)),
            vi((EBUF,)), vi((EBUF,)), vi((EBUF,)), vi((EBUF,)), vi((EBUF,)),
            vi((CH,)), vi((CH,)), vi((CH,)), vi((CH,)), vi((CH,)),
            vf((3, CH, D)), vf((CH, D)), vf((CH, D)), vf((CH, D)),
            pltpu.SemaphoreType.DMA,
        ])
    return k(x_sum, emb2, eat_t, edge_index, added_sym_edge)


def _scatter_sc_kernel(msg_hbm, ei_hbm, part_hbm, cntp_hbm,
                       colf, idxb, msgb, ones_v, zrow, zcnt,
                       acc_sh, cnt_sh, sem):
    sid = lax.axis_index("s")
    cid = lax.axis_index("c")
    w = sid * 2 + cid
    lo = _chunk_start(w) * CH

    zero16 = jnp.zeros((16,), jnp.float32)
    one16 = jnp.full((16,), 1.0, jnp.float32)

    def zr(r, _):
        for k in range(D // 16):
            zrow[r, pl.ds(k * 16, 16)] = zero16
        return 0
    lax.fori_loop(0, ZR, zr, 0)

    def zc(r, _):
        zcnt[r, pl.ds(0, 16)] = zero16
        return 0
    lax.fori_loop(0, 312, zc, 0)

    def ov(r, _):
        ones_v[r, pl.ds(0, 16)] = one16
        return 0
    lax.fori_loop(0, CH, ov, 0)

    pltpu.sync_copy(ei_hbm.at[pl.ds(E + lo, CPT * CH)], colf.at[pl.ds(0, CPT * CH)])

    @pl.when(w < XTRA)
    def _():
        pltpu.sync_copy(ei_hbm.at[pl.ds(E + lo + CPT * CH, CH)],
                        colf.at[pl.ds(CPT * CH, CH)])

    for p in range(2):
        nb = p * PH
        pltpu.sync_copy(zrow, acc_sh.at[pl.ds(sid * 312, ZR)])
        pltpu.sync_copy(zrow, acc_sh.at[pl.ds(sid * 312 + ZR, ZR)])
        pltpu.sync_copy(zcnt, cnt_sh.at[pl.ds(sid * 312, 312)])

        @pl.when(sid == 15)
        def _():
            pltpu.sync_copy(zrow.at[pl.ds(0, PACC - 4992)],
                            acc_sh.at[pl.ds(4992, PACC - 4992)])
            pltpu.sync_copy(zcnt.at[pl.ds(0, PACC - 4992)],
                            cnt_sh.at[pl.ds(4992, PACC - 4992)])

        plsc.subcore_barrier()

        def chunk(j, _):
            lb = j * CH
            for q in range(CH // 16):
                loc = colf[pl.ds(lb + q * 16, 16)] - nb
                m = (loc >= 0) & (loc < PH)
                idxb[pl.ds(q * 16, 16)] = jnp.where(m, loc, PH)
            pltpu.sync_copy(msg_hbm.at[pl.ds(lo + lb, CH)], msgb)
            pltpu.sync_copy(msgb, acc_sh.at[idxb], add=True)
            pltpu.sync_copy(ones_v, cnt_sh.at[idxb], add=True)
            return 0

        lax.fori_loop(0, CPT, chunk, 0)

        @pl.when(w < XTRA)
        def _():
            chunk(CPT, 0)

        plsc.subcore_barrier()
        sl = pl.ds(sid * 312, 312)
        glb = pl.ds(nb + sid * 312, 312)
        pltpu.sync_copy(acc_sh.at[sl], part_hbm.at[cid, glb])
        pltpu.sync_copy(cnt_sh.at[sl], cntp_hbm.at[cid, glb])

        @pl.when(sid == 15)
        def _():
            sl2 = pl.ds(4992, 8)
            glb2 = pl.ds(nb + 4992, 8)
            pltpu.sync_copy(acc_sh.at[sl2], part_hbm.at[cid, glb2])
            pltpu.sync_copy(cnt_sh.at[sl2], cntp_hbm.at[cid, glb2])


def _scatter_sc(msg, edge_index_flat):
    vf = lambda shape: pltpu.VMEM(shape, jnp.float32)
    k = pl.kernel(
        _scatter_sc_kernel, mesh=_mesh(),
        compiler_params=pltpu.CompilerParams(needs_layout_passes=False,
                                             use_tc_tiling_on_sc=False),
        out_type=[jax.ShapeDtypeStruct((2, N, D), jnp.float32),
                  jax.ShapeDtypeStruct((2, N, 16), jnp.float32)],
        scratch_types=[
            pltpu.VMEM((EBUF,), jnp.int32), pltpu.VMEM((CH,), jnp.int32),
            vf((CH, D)), vf((CH, 16)), vf((ZR, D)), vf((312, 16)),
            pltpu.VMEM_SHARED((PACC, D), jnp.float32),
            pltpu.VMEM_SHARED((PACC, 16), jnp.float32),
            pltpu.SemaphoreType.DMA,
        ])
    return k(msg, edge_index_flat)


def _edge_block_kernel(xr_ref, xc_ref, ea_ref,
                       ew1_ref, eb1_ref, ew2_ref, eb2_ref,
                       nw1_ref, nb1_ref, nw2_ref, nb2_ref,
                       eout_ref, msg_ref):
    xr = xr_ref[...]
    xc = xc_ref[...]
    ea = ea_ref[...]
    w_r = ew1_ref[0:D, :]
    w_c = ew1_ref[D:2 * D, :]
    w_a = ew1_ref[2 * D:3 * D, :]
    h = jnp.dot(xr, w_r, preferred_element_type=jnp.float32)
    h += jnp.dot(xc, w_c, preferred_element_type=jnp.float32)
    h += jnp.dot(ea, w_a, preferred_element_type=jnp.float32)
    h = jnp.maximum(h + eb1_ref[...], 0.0)
    eout = jnp.dot(h, ew2_ref[...], preferred_element_type=jnp.float32) + eb2_ref[...]
    eout_ref[...] = eout

    m_r = nw1_ref[0:D, :]
    m_e = nw1_ref[D:2 * D, :]
    m = jnp.dot(xr, m_r, preferred_element_type=jnp.float32)
    m += jnp.dot(eout, m_e, preferred_element_type=jnp.float32)
    m = jnp.maximum(m + nb1_ref[...], 0.0)
    msg_ref[...] = jnp.dot(m, nw2_ref[...], preferred_element_type=jnp.float32) + nb2_ref[...]


def _edge_mlps(xr, xc, ea, e_w1, e_b1, e_w2, e_b2, n1_w1, n1_b1, n1_w2, n1_b2):
    nblk = E // BE
    full = lambda shape: pl.BlockSpec(shape, lambda i: (0,) * len(shape))
    blk = pl.BlockSpec((BE, D), lambda i: (i, 0))
    return pl.pallas_call(
        _edge_block_kernel,
        grid=(nblk,),
        in_specs=[blk, blk, blk,
                  full((3 * D, D)), full((1, D)), full((D, D)), full((1, D)),
                  full((2 * D, D)), full((1, D)), full((D, D)), full((1, D))],
        out_specs=[blk, blk],
        out_shape=[jax.ShapeDtypeStruct((E, D), jnp.float32),
                   jax.ShapeDtypeStruct((E, D), jnp.float32)],
    )(xr, xc, ea, e_w1, e_b1, e_w2, e_b2, n1_w1, n1_b1, n1_w2, n1_b2)


def _node_block_kernel(xs_ref, part_ref, cntp_ref, batch_row_ref,
                       w1_ref, b1_ref, w2_ref, b2_ref,
                       xout_ref, stats_ref):
    i = pl.program_id(0)
    xs = xs_ref[...]
    cnt = (cntp_ref[0] + cntp_ref[1])[:, 0:1]
    agg = (part_ref[0] + part_ref[1]) / jnp.maximum(cnt, 1.0)
    w_x = w1_ref[0:D, :]
    w_a = w1_ref[D:2 * D, :]
    h = jnp.dot(xs, w_x, preferred_element_type=jnp.float32)
    h += jnp.dot(agg, w_a, preferred_element_type=jnp.float32)
    h = jnp.maximum(h + b1_ref[...], 0.0)
    xo = jnp.dot(h, w2_ref[...], preferred_element_type=jnp.float32) + b2_ref[...]
    xout_ref[...] = xo

    gids = lax.broadcasted_iota(jnp.int32, (NG, BN), 0)
    onehot_t = (gids == batch_row_ref[0]).astype(jnp.float32)
    cat = jnp.concatenate([xo, xo * xo, jnp.ones((BN, D), jnp.float32)], axis=1)
    part = jnp.dot(onehot_t, cat, preferred_element_type=jnp.float32)

    @pl.when(i == 0)
    def _():
        stats_ref[...] = jnp.zeros_like(stats_ref)

    stats_ref[...] += part


def _node_mlp_stats(x_sum, part, cntp, batch_row, n2_w1, n2_b1, n2_w2, n2_b2):
    nblk = N // BN
    full = lambda shape: pl.BlockSpec(shape, lambda i: (0,) * len(shape))
    blk = pl.BlockSpec((BN, D), lambda i: (i, 0))
    return pl.pallas_call(
        _node_block_kernel,
        grid=(nblk,),
        in_specs=[blk, pl.BlockSpec((2, BN, D), lambda i: (0, i, 0)),
                  pl.BlockSpec((2, BN, 16), lambda i: (0, i, 0)),
                  pl.BlockSpec((1, 1, BN), lambda i: (i, 0, 0)),
                  full((2 * D, D)), full((1, D)), full((D, D)), full((1, D))],
        out_specs=[blk, full((NG, 3 * D))],
        out_shape=[jax.ShapeDtypeStruct((N, D), jnp.float32),
                   jax.ShapeDtypeStruct((NG, 3 * D), jnp.float32)],
    )(x_sum, part, cntp, batch_row, n2_w1, n2_b1, n2_w2, n2_b2)


def _norm_block_kernel(xo_ref, batch_col_ref, stats_ref, gamma_ref, beta_ref, out_ref):
    xo = xo_ref[...]
    gids = lax.broadcasted_iota(jnp.int32, (BN, NG), 1)
    onehot = (gids == batch_col_ref[...]).astype(jnp.float32)
    st = jnp.dot(onehot, stats_ref[...], preferred_element_type=jnp.float32)
    cnt = jnp.maximum(st[:, 2 * D:2 * D + 1], 1.0)
    mean = st[:, 0:D] / cnt
    var = st[:, D:2 * D] / cnt - mean * mean
    inv = lax.rsqrt(jnp.maximum(var, 0.0) + 1e-5)
    out_ref[...] = (xo - mean) * inv * gamma_ref[...] + beta_ref[...]


def _graph_norm(x_out, batch_col, stats, gamma, beta):
    nblk = N // BN
    full = lambda shape: pl.BlockSpec(shape, lambda i: (0,) * len(shape))
    blk = pl.BlockSpec((BN, D), lambda i: (i, 0))
    return pl.pallas_call(
        _norm_block_kernel,
        grid=(nblk,),
        in_specs=[blk, pl.BlockSpec((BN, 1), lambda i: (i, 0)),
                  full((NG, 3 * D)), full((1, D)), full((1, D))],
        out_specs=blk,
        out_shape=jax.ShapeDtypeStruct((N, D), jnp.float32),
    )(x_out, batch_col, stats, gamma, beta)


def kernel(x_tokens, edge_index, edge_attr_tokens, added_sym_edge, batch, emb,
           e_w1, e_b1, e_w2, e_b2, n1_w1, n1_b1, n1_w2, n1_b2,
           n2_w1, n2_b1, n2_w2, n2_b2, gamma, beta):
    x_tokens = x_tokens.astype(jnp.int32)
    edge_attr_tokens = edge_attr_tokens.astype(jnp.int32)
    edge_index = edge_index.astype(jnp.int32)
    added_sym_edge = added_sym_edge.astype(jnp.int32)
    batch = batch.astype(jnp.int32)

    x_sum = _xsum_sc(emb, jnp.transpose(x_tokens).reshape(-1))

    emb2 = jnp.concatenate([emb, -emb], axis=0)
    xr, xc, ea_sum = _edge_sc(x_sum, emb2,
                              jnp.transpose(edge_attr_tokens).reshape(-1),
                              edge_index.reshape(-1), added_sym_edge)

    r2 = lambda v: v.reshape(1, D)
    edge_out, msg = _edge_mlps(xr, xc, ea_sum, e_w1, r2(e_b1), e_w2, r2(e_b2),
                               n1_w1, r2(n1_b1), n1_w2, r2(n1_b2))

    part, cntp = _scatter_sc(msg, edge_index.reshape(-1))

    batch_row = batch.reshape(N // BN, 1, BN)
    x_out, stats = _node_mlp_stats(x_sum, part, cntp, batch_row,
                                   n2_w1, r2(n2_b1), n2_w2, r2(n2_b2))

    x_norm = _graph_norm(x_out, batch.reshape(N, 1), stats, r2(gamma), r2(beta))
    return (x_norm, edge_out)


# speedup vs baseline: 5.7638x; 1.1444x over previous
import functools

import jax
import jax.numpy as jnp
from jax import lax
from jax.experimental import pallas as pl
from jax.experimental.pallas import tpu as pltpu
from jax.experimental.pallas import tpu_sc as plsc

N = 10000
E = 160000
D = 128
VOCAB = 3000
TOK = 3
NG = 128
NSYM = 80000

BE = 2000
BN = 2000

NW = 32
CH = 64
ECH = E // CH
CPT = ECH // NW
XTRA = ECH - CPT * NW
EBUF = (CPT + 1) * CH
ZR = 156
PH = 5000
PACC = 5008

def _mesh():
    return plsc.VectorSubcoreMesh(core_axis_name="c", subcore_axis_name="s")


def _wid():
    return lax.axis_index("s") * 2 + lax.axis_index("c")


def _chunk_start(w):
    return w * CPT + jnp.minimum(w, XTRA)


def _vec_sum3(rows3, out, nrows):
    def body(r, _):
        for k in range(D // 16):
            s = pl.ds(k * 16, 16)
            out[r, s] = rows3[0, r, s] + rows3[1, r, s] + rows3[2, r, s]
        return 0
    lax.fori_loop(0, nrows, body, 0)


def _xsum_sc_kernel(emb_hbm, xtok_hbm, out_hbm, t0, t1, t2, rows3, ob, sem):
    w = _wid()
    nch = (N + CH - 1) // CH

    def chunk(j, _):
        t = w + j * NW

        @pl.when(t < nch)
        def _():
            base = jnp.minimum(t * CH, N - CH)
            pltpu.sync_copy(xtok_hbm.at[pl.ds(base, CH)], t0)
            pltpu.sync_copy(xtok_hbm.at[pl.ds(N + base, CH)], t1)
            pltpu.sync_copy(xtok_hbm.at[pl.ds(2 * N + base, CH)], t2)
            c0 = pltpu.async_copy(emb_hbm.at[t0], rows3.at[0], sem)
            c1 = pltpu.async_copy(emb_hbm.at[t1], rows3.at[1], sem)
            c2 = pltpu.async_copy(emb_hbm.at[t2], rows3.at[2], sem)
            c0.wait(); c1.wait(); c2.wait()
            _vec_sum3(rows3, ob, CH)
            pltpu.sync_copy(ob, out_hbm.at[pl.ds(base, CH)])
        return 0

    lax.fori_loop(0, (nch + NW - 1) // NW, chunk, 0)


def _xsum_sc(emb, xtok_t):
    k = pl.kernel(
        _xsum_sc_kernel, mesh=_mesh(),
        compiler_params=pltpu.CompilerParams(needs_layout_passes=False),
        out_type=jax.ShapeDtypeStruct((N, D), jnp.float32),
        scratch_types=[
            pltpu.VMEM((CH,), jnp.int32), pltpu.VMEM((CH,), jnp.int32),
            pltpu.VMEM((CH,), jnp.int32),
            pltpu.VMEM((3, CH, D), jnp.float32),
            pltpu.VMEM((CH, D), jnp.float32),
            pltpu.SemaphoreType.DMA,
        ])
    return k(emb, xtok_t)


SYB = 1600


def _edge_sc_kernel(xsum_hbm, emb2_hbm, eat_hbm, ei_hbm, sym_hbm,
                    xr_hbm, xc_hbm, ea_hbm,
                    mask_v, symb, rowf, colf, tk0, tk1, tk2,
                    ixr0, ixc0, it00, it10, it20, rows30, xrb0, xcb0,
                    ixr1, ixc1, it01, it11, it21, rows31, xrb1, xcb1,
                    sg0, sg1, sw0, sw1):
    w = _wid()
    c_lo = _chunk_start(w)
    c_hi = _chunk_start(w + 1)
    lo = c_lo * CH
    n_ch = c_hi - c_lo
    n_loc = n_ch * CH

    for hbm, off, buf in ((ei_hbm, 0, rowf), (ei_hbm, E, colf),
                          (eat_hbm, 0, tk0), (eat_hbm, E, tk1),
                          (eat_hbm, 2 * E, tk2)):
        pltpu.sync_copy(hbm.at[pl.ds(off + lo, CPT * CH)],
                        buf.at[pl.ds(0, CPT * CH)])

        @pl.when(w < XTRA)
        def _():
            pltpu.sync_copy(hbm.at[pl.ds(off + lo + CPT * CH, CH)],
                            buf.at[pl.ds(CPT * CH, CH)])

    zero16 = jnp.zeros((16,), jnp.int32)
    one16 = jnp.full((16,), 1, jnp.int32)

    def zeroit(q, _):
        mask_v[pl.ds(q * 16, 16)] = zero16
        return 0
    lax.fori_loop(0, EBUF // 16, zeroit, 0)

    def symscan(s0, _):
        pltpu.sync_copy(sym_hbm.at[pl.ds(s0 * SYB, SYB)], symb)

        def inner(q, _):
            v = symb[pl.ds(q * 16, 16)]
            loc = v - lo
            m = (loc >= 0) & (loc < n_loc)
            locc = jnp.clip(loc, 0, EBUF - 1)
            plsc.store_scatter(mask_v, [locc], one16, mask=m)
            return 0
        lax.fori_loop(0, SYB // 16, inner, 0)
        return 0
    lax.fori_loop(0, NSYM // SYB, symscan, 0)

    sets = ((ixr0, ixc0, it00, it10, it20, rows30, xrb0, xcb0, sg0, sw0),
            (ixr1, ixc1, it01, it11, it21, rows31, xrb1, xcb1, sg1, sw1))

    def drain_writes(s):
        _, _, _, _, _, rows3, xrb, xcb, _, sw = s
        pltpu.make_async_copy(xr_hbm.at[pl.ds(0, CH)], xrb, sw).wait()
        pltpu.make_async_copy(xr_hbm.at[pl.ds(0, CH)], xcb, sw).wait()
        pltpu.make_async_copy(ea_hbm.at[pl.ds(0, CH)], rows3.at[0], sw).wait()

    def fire(j, s):
        ixr, ixc, it0, it1, it2, rows3, xrb, xcb, sg, _ = s
        lb = j * CH
        for q in range(CH // 16):
            sl = pl.ds(lb + q * 16, 16)
            so = pl.ds(q * 16, 16)
            mv = mask_v[sl] * VOCAB
            it0[so] = tk0[sl] + mv
            it1[so] = tk1[sl] + mv
            it2[so] = tk2[sl] + mv
            ixr[so] = rowf[sl]
            ixc[so] = colf[sl]
        pltpu.async_copy(xsum_hbm.at[ixr], xrb, sg)
        pltpu.async_copy(xsum_hbm.at[ixc], xcb, sg)
        pltpu.async_copy(emb2_hbm.at[it0], rows3.at[0], sg)
        pltpu.async_copy(emb2_hbm.at[it1], rows3.at[1], sg)
        pltpu.async_copy(emb2_hbm.at[it2], rows3.at[2], sg)

    def complete(j, s):
        _, _, _, _, _, rows3, xrb, xcb, sg, sw = s
        pltpu.make_async_copy(xsum_hbm.at[pl.ds(0, CH)], xrb, sg).wait()
        pltpu.make_async_copy(xsum_hbm.at[pl.ds(0, CH)], xcb, sg).wait()
        for t in range(3):
            pltpu.make_async_copy(emb2_hbm.at[pl.ds(0, CH)], rows3.at[t], sg).wait()

        def body(r, _):
            for k in range(D // 16):
                sk = pl.ds(k * 16, 16)
                rows3[0, r, sk] = rows3[0, r, sk] + rows3[1, r, sk] + rows3[2, r, sk]
            return 0
        lax.fori_loop(0, CH, body, 0)
        g = lo + j * CH
        pltpu.async_copy(xrb, xr_hbm.at[pl.ds(g, CH)], sw)
        pltpu.async_copy(xcb, xc_hbm.at[pl.ds(g, CH)], sw)
        pltpu.async_copy(rows3.at[0], ea_hbm.at[pl.ds(g, CH)], sw)

    def pair(jj, _):
        j0 = jj * 2
        j1 = jj * 2 + 1

        @pl.when(j0 < n_ch)
        def _():
            @pl.when(j0 >= 2)
            def _():
                drain_writes(sets[0])
            fire(j0, sets[0])

        @pl.when((j0 >= 1) & (j0 - 1 < n_ch))
        def _():
            complete(j0 - 1, sets[1])

        @pl.when(j1 < n_ch)
        def _():
            @pl.when(j1 >= 2)
            def _():
                drain_writes(sets[1])
            fire(j1, sets[1])

        @pl.when(j1 - 1 < n_ch)
        def _():
            complete(j1 - 1, sets[0])
        return 0

    lax.fori_loop(0, (CPT + 2) // 2, pair, 0)

    drain_writes(sets[0])
    drain_writes(sets[1])


def _edge_sc(x_sum, emb2, eat_t, edge_index, added_sym_edge):
    vi = lambda shape: pltpu.VMEM(shape, jnp.int32)
    vf = lambda shape: pltpu.VMEM(shape, jnp.float32)
    bufset = [vi((CH,))] * 5 + [vf((3, CH, D)), vf((CH, D)), vf((CH, D))]
    k = pl.kernel(
        _edge_sc_kernel, mesh=_mesh(),
        compiler_params=pltpu.CompilerParams(needs_layout_passes=False),
        out_type=[jax.ShapeDtypeStruct((E, D), jnp.float32)] * 3,
        scratch_types=[
            vi((EBUF,)), vi((SYB,)),
            vi((EBUF,)), vi((EBUF,)), vi((EBUF,)), vi((EBUF,)), vi((EBUF,)),
        ] + bufset + bufset + [
            pltpu.SemaphoreType.DMA, pltpu.SemaphoreType.DMA,
            pltpu.SemaphoreType.DMA, pltpu.SemaphoreType.DMA,
        ])
    return k(x_sum, emb2, eat_t, edge_index, added_sym_edge)


def _scatter_sc_kernel(msg_hbm, ei_hbm, part_hbm, cntp_hbm,
                       colf, idxb, msgb, ones_v, zrow, zcnt,
                       acc_sh, cnt_sh, sem):
    sid = lax.axis_index("s")
    cid = lax.axis_index("c")
    w = sid * 2 + cid
    lo = _chunk_start(w) * CH

    zero16 = jnp.zeros((16,), jnp.float32)
    one16 = jnp.full((16,), 1.0, jnp.float32)

    def zr(r, _):
        for k in range(D // 16):
            zrow[r, pl.ds(k * 16, 16)] = zero16
        return 0
    lax.fori_loop(0, ZR, zr, 0)

    def zc(r, _):
        zcnt[r, pl.ds(0, 16)] = zero16
        return 0
    lax.fori_loop(0, 312, zc, 0)

    def ov(r, _):
        ones_v[r, pl.ds(0, 16)] = one16
        return 0
    lax.fori_loop(0, CH, ov, 0)

    pltpu.sync_copy(ei_hbm.at[pl.ds(E + lo, CPT * CH)], colf.at[pl.ds(0, CPT * CH)])

    @pl.when(w < XTRA)
    def _():
        pltpu.sync_copy(ei_hbm.at[pl.ds(E + lo + CPT * CH, CH)],
                        colf.at[pl.ds(CPT * CH, CH)])

    for p in range(2):
        nb = p * PH
        pltpu.sync_copy(zrow, acc_sh.at[pl.ds(sid * 312, ZR)])
        pltpu.sync_copy(zrow, acc_sh.at[pl.ds(sid * 312 + ZR, ZR)])
        pltpu.sync_copy(zcnt, cnt_sh.at[pl.ds(sid * 312, 312)])

        @pl.when(sid == 15)
        def _():
            pltpu.sync_copy(zrow.at[pl.ds(0, PACC - 4992)],
                            acc_sh.at[pl.ds(4992, PACC - 4992)])
            pltpu.sync_copy(zcnt.at[pl.ds(0, PACC - 4992)],
                            cnt_sh.at[pl.ds(4992, PACC - 4992)])

        plsc.subcore_barrier()

        def chunk(j, _):
            lb = j * CH
            for q in range(CH // 16):
                loc = colf[pl.ds(lb + q * 16, 16)] - nb
                m = (loc >= 0) & (loc < PH)
                idxb[pl.ds(q * 16, 16)] = jnp.where(m, loc, PH)
            pltpu.sync_copy(msg_hbm.at[pl.ds(lo + lb, CH)], msgb)
            pltpu.sync_copy(msgb, acc_sh.at[idxb], add=True)
            pltpu.sync_copy(ones_v, cnt_sh.at[idxb], add=True)
            return 0

        lax.fori_loop(0, CPT, chunk, 0)

        @pl.when(w < XTRA)
        def _():
            chunk(CPT, 0)

        plsc.subcore_barrier()
        sl = pl.ds(sid * 312, 312)
        glb = pl.ds(nb + sid * 312, 312)
        pltpu.sync_copy(acc_sh.at[sl], part_hbm.at[cid, glb])
        pltpu.sync_copy(cnt_sh.at[sl], cntp_hbm.at[cid, glb])

        @pl.when(sid == 15)
        def _():
            sl2 = pl.ds(4992, 8)
            glb2 = pl.ds(nb + 4992, 8)
            pltpu.sync_copy(acc_sh.at[sl2], part_hbm.at[cid, glb2])
            pltpu.sync_copy(cnt_sh.at[sl2], cntp_hbm.at[cid, glb2])


def _scatter_sc(msg, edge_index_flat):
    vf = lambda shape: pltpu.VMEM(shape, jnp.float32)
    k = pl.kernel(
        _scatter_sc_kernel, mesh=_mesh(),
        compiler_params=pltpu.CompilerParams(needs_layout_passes=False,
                                             use_tc_tiling_on_sc=False),
        out_type=[jax.ShapeDtypeStruct((2, N, D), jnp.float32),
                  jax.ShapeDtypeStruct((2, N, 16), jnp.float32)],
        scratch_types=[
            pltpu.VMEM((EBUF,), jnp.int32), pltpu.VMEM((CH,), jnp.int32),
            vf((CH, D)), vf((CH, 16)), vf((ZR, D)), vf((312, 16)),
            pltpu.VMEM_SHARED((PACC, D), jnp.float32),
            pltpu.VMEM_SHARED((PACC, 16), jnp.float32),
            pltpu.SemaphoreType.DMA,
        ])
    return k(msg, edge_index_flat)


def _edge_block_kernel(xr_ref, xc_ref, ea_ref,
                       ew1_ref, eb1_ref, ew2_ref, eb2_ref,
                       nw1_ref, nb1_ref, nw2_ref, nb2_ref,
                       eout_ref, msg_ref):
    xr = xr_ref[...]
    xc = xc_ref[...]
    ea = ea_ref[...]
    w_r = ew1_ref[0:D, :]
    w_c = ew1_ref[D:2 * D, :]
    w_a = ew1_ref[2 * D:3 * D, :]
    h = jnp.dot(xr, w_r, preferred_element_type=jnp.float32)
    h += jnp.dot(xc, w_c, preferred_element_type=jnp.float32)
    h += jnp.dot(ea, w_a, preferred_element_type=jnp.float32)
    h = jnp.maximum(h + eb1_ref[...], 0.0)
    eout = jnp.dot(h, ew2_ref[...], preferred_element_type=jnp.float32) + eb2_ref[...]
    eout_ref[...] = eout

    m_r = nw1_ref[0:D, :]
    m_e = nw1_ref[D:2 * D, :]
    m = jnp.dot(xr, m_r, preferred_element_type=jnp.float32)
    m += jnp.dot(eout, m_e, preferred_element_type=jnp.float32)
    m = jnp.maximum(m + nb1_ref[...], 0.0)
    msg_ref[...] = jnp.dot(m, nw2_ref[...], preferred_element_type=jnp.float32) + nb2_ref[...]


def _edge_mlps(xr, xc, ea, e_w1, e_b1, e_w2, e_b2, n1_w1, n1_b1, n1_w2, n1_b2):
    nblk = E // BE
    full = lambda shape: pl.BlockSpec(shape, lambda i: (0,) * len(shape))
    blk = pl.BlockSpec((BE, D), lambda i: (i, 0))
    return pl.pallas_call(
        _edge_block_kernel,
        grid=(nblk,),
        in_specs=[blk, blk, blk,
                  full((3 * D, D)), full((1, D)), full((D, D)), full((1, D)),
                  full((2 * D, D)), full((1, D)), full((D, D)), full((1, D))],
        out_specs=[blk, blk],
        out_shape=[jax.ShapeDtypeStruct((E, D), jnp.float32),
                   jax.ShapeDtypeStruct((E, D), jnp.float32)],
    )(xr, xc, ea, e_w1, e_b1, e_w2, e_b2, n1_w1, n1_b1, n1_w2, n1_b2)


def _node_block_kernel(xs_ref, part_ref, cntp_ref, batch_row_ref,
                       w1_ref, b1_ref, w2_ref, b2_ref,
                       xout_ref, stats_ref):
    i = pl.program_id(0)
    xs = xs_ref[...]
    cnt = (cntp_ref[0] + cntp_ref[1])[:, 0:1]
    agg = (part_ref[0] + part_ref[1]) / jnp.maximum(cnt, 1.0)
    w_x = w1_ref[0:D, :]
    w_a = w1_ref[D:2 * D, :]
    h = jnp.dot(xs, w_x, preferred_element_type=jnp.float32)
    h += jnp.dot(agg, w_a, preferred_element_type=jnp.float32)
    h = jnp.maximum(h + b1_ref[...], 0.0)
    xo = jnp.dot(h, w2_ref[...], preferred_element_type=jnp.float32) + b2_ref[...]
    xout_ref[...] = xo

    gids = lax.broadcasted_iota(jnp.int32, (NG, BN), 0)
    onehot_t = (gids == batch_row_ref[0]).astype(jnp.float32)
    cat = jnp.concatenate([xo, xo * xo, jnp.ones((BN, D), jnp.float32)], axis=1)
    part = jnp.dot(onehot_t, cat, preferred_element_type=jnp.float32)

    @pl.when(i == 0)
    def _():
        stats_ref[...] = jnp.zeros_like(stats_ref)

    stats_ref[...] += part


def _node_mlp_stats(x_sum, part, cntp, batch_row, n2_w1, n2_b1, n2_w2, n2_b2):
    nblk = N // BN
    full = lambda shape: pl.BlockSpec(shape, lambda i: (0,) * len(shape))
    blk = pl.BlockSpec((BN, D), lambda i: (i, 0))
    return pl.pallas_call(
        _node_block_kernel,
        grid=(nblk,),
        in_specs=[blk, pl.BlockSpec((2, BN, D), lambda i: (0, i, 0)),
                  pl.BlockSpec((2, BN, 16), lambda i: (0, i, 0)),
                  pl.BlockSpec((1, 1, BN), lambda i: (i, 0, 0)),
                  full((2 * D, D)), full((1, D)), full((D, D)), full((1, D))],
        out_specs=[blk, full((NG, 3 * D))],
        out_shape=[jax.ShapeDtypeStruct((N, D), jnp.float32),
                   jax.ShapeDtypeStruct((NG, 3 * D), jnp.float32)],
    )(x_sum, part, cntp, batch_row, n2_w1, n2_b1, n2_w2, n2_b2)


def _norm_block_kernel(xo_ref, batch_col_ref, stats_ref, gamma_ref, beta_ref, out_ref):
    xo = xo_ref[...]
    gids = lax.broadcasted_iota(jnp.int32, (BN, NG), 1)
    onehot = (gids == batch_col_ref[...]).astype(jnp.float32)
    st = jnp.dot(onehot, stats_ref[...], preferred_element_type=jnp.float32)
    cnt = jnp.maximum(st[:, 2 * D:2 * D + 1], 1.0)
    mean = st[:, 0:D] / cnt
    var = st[:, D:2 * D] / cnt - mean * mean
    inv = lax.rsqrt(jnp.maximum(var, 0.0) + 1e-5)
    out_ref[...] = (xo - mean) * inv * gamma_ref[...] + beta_ref[...]


def _graph_norm(x_out, batch_col, stats, gamma, beta):
    nblk = N // BN
    full = lambda shape: pl.BlockSpec(shape, lambda i: (0,) * len(shape))
    blk = pl.BlockSpec((BN, D), lambda i: (i, 0))
    return pl.pallas_call(
        _norm_block_kernel,
        grid=(nblk,),
        in_specs=[blk, pl.BlockSpec((BN, 1), lambda i: (i, 0)),
                  full((NG, 3 * D)), full((1, D)), full((1, D))],
        out_specs=blk,
        out_shape=jax.ShapeDtypeStruct((N, D), jnp.float32),
    )(x_out, batch_col, stats, gamma, beta)


def kernel(x_tokens, edge_index, edge_attr_tokens, added_sym_edge, batch, emb,
           e_w1, e_b1, e_w2, e_b2, n1_w1, n1_b1, n1_w2, n1_b2,
           n2_w1, n2_b1, n2_w2, n2_b2, gamma, beta):
    x_tokens = x_tokens.astype(jnp.int32)
    edge_attr_tokens = edge_attr_tokens.astype(jnp.int32)
    edge_index = edge_index.astype(jnp.int32)
    added_sym_edge = added_sym_edge.astype(jnp.int32)
    batch = batch.astype(jnp.int32)

    x_sum = _xsum_sc(emb, jnp.transpose(x_tokens).reshape(-1))

    emb2 = jnp.concatenate([emb, -emb], axis=0)
    xr, xc, ea_sum = _edge_sc(x_sum, emb2,
                              jnp.transpose(edge_attr_tokens).reshape(-1),
                              edge_index.reshape(-1), added_sym_edge)

    r2 = lambda v: v.reshape(1, D)
    edge_out, msg = _edge_mlps(xr, xc, ea_sum, e_w1, r2(e_b1), e_w2, r2(e_b2),
                               n1_w1, r2(n1_b1), n1_w2, r2(n1_b2))

    part, cntp = _scatter_sc(msg, edge_index.reshape(-1))

    batch_row = batch.reshape(N // BN, 1, BN)
    x_out, stats = _node_mlp_stats(x_sum, part, cntp, batch_row,
                                   n2_w1, r2(n2_b1), n2_w2, r2(n2_b2))

    x_norm = _graph_norm(x_out, batch.reshape(N, 1), stats, r2(gamma), r2(beta))
    return (x_norm, edge_out)


# speedup vs baseline: 6.3623x; 1.1038x over previous
import functools

import jax
import jax.numpy as jnp
from jax import lax
from jax.experimental import pallas as pl
from jax.experimental.pallas import tpu as pltpu
from jax.experimental.pallas import tpu_sc as plsc

N = 10000
E = 160000
D = 128
VOCAB = 3000
TOK = 3
NG = 128
NSYM = 80000

BE = 2000
BN = 2000

NW = 32
CH = 64
ECH = E // CH
CPT = ECH // NW
XTRA = ECH - CPT * NW
EBUF = (CPT + 1) * CH
ZR = 156
PH = 5000
PACC = 5008

def _mesh():
    return plsc.VectorSubcoreMesh(core_axis_name="c", subcore_axis_name="s")


def _wid():
    return lax.axis_index("s") * 2 + lax.axis_index("c")


def _chunk_start(w):
    return w * CPT + jnp.minimum(w, XTRA)


def _vec_sum3(rows3, out, nrows):
    def body(r, _):
        for k in range(D // 16):
            s = pl.ds(k * 16, 16)
            out[r, s] = rows3[0, r, s] + rows3[1, r, s] + rows3[2, r, s]
        return 0
    lax.fori_loop(0, nrows, body, 0)


def _xsum_sc_kernel(emb_hbm, xtok_hbm, out_hbm, t0, t1, t2, rows3, ob, sem):
    w = _wid()
    nch = (N + CH - 1) // CH

    def chunk(j, _):
        t = w + j * NW

        @pl.when(t < nch)
        def _():
            base = jnp.minimum(t * CH, N - CH)
            pltpu.sync_copy(xtok_hbm.at[pl.ds(base, CH)], t0)
            pltpu.sync_copy(xtok_hbm.at[pl.ds(N + base, CH)], t1)
            pltpu.sync_copy(xtok_hbm.at[pl.ds(2 * N + base, CH)], t2)
            c0 = pltpu.async_copy(emb_hbm.at[t0], rows3.at[0], sem)
            c1 = pltpu.async_copy(emb_hbm.at[t1], rows3.at[1], sem)
            c2 = pltpu.async_copy(emb_hbm.at[t2], rows3.at[2], sem)
            c0.wait(); c1.wait(); c2.wait()
            _vec_sum3(rows3, ob, CH)
            pltpu.sync_copy(ob, out_hbm.at[pl.ds(base, CH)])
        return 0

    lax.fori_loop(0, (nch + NW - 1) // NW, chunk, 0)


def _xsum_sc(emb, xtok_t):
    k = pl.kernel(
        _xsum_sc_kernel, mesh=_mesh(),
        compiler_params=pltpu.CompilerParams(needs_layout_passes=False),
        out_type=jax.ShapeDtypeStruct((N, D), jnp.float32),
        scratch_types=[
            pltpu.VMEM((CH,), jnp.int32), pltpu.VMEM((CH,), jnp.int32),
            pltpu.VMEM((CH,), jnp.int32),
            pltpu.VMEM((3, CH, D), jnp.float32),
            pltpu.VMEM((CH, D), jnp.float32),
            pltpu.SemaphoreType.DMA,
        ])
    return k(emb, xtok_t)


SYB = 1600


def _edge_sc_kernel(xsum_hbm, emb2_hbm, eat_hbm, ei_hbm, sym_hbm,
                    xr_hbm, xc_hbm, ea_hbm,
                    mask_v, symb, rowf, colf, tk0, tk1, tk2,
                    ixr0, ixc0, it00, it10, it20, rows30, xrb0, xcb0,
                    ixr1, ixc1, it01, it11, it21, rows31, xrb1, xcb1,
                    sg0, sg1, sw0, sw1):
    w = _wid()
    c_lo = _chunk_start(w)
    c_hi = _chunk_start(w + 1)
    lo = c_lo * CH
    n_ch = c_hi - c_lo
    n_loc = n_ch * CH

    for hbm, off, buf in ((ei_hbm, 0, rowf), (ei_hbm, E, colf),
                          (eat_hbm, 0, tk0), (eat_hbm, E, tk1),
                          (eat_hbm, 2 * E, tk2)):
        pltpu.sync_copy(hbm.at[pl.ds(off + lo, CPT * CH)],
                        buf.at[pl.ds(0, CPT * CH)])

        @pl.when(w < XTRA)
        def _():
            pltpu.sync_copy(hbm.at[pl.ds(off + lo + CPT * CH, CH)],
                            buf.at[pl.ds(CPT * CH, CH)])

    zero16 = jnp.zeros((16,), jnp.int32)
    one16 = jnp.full((16,), 1, jnp.int32)

    def zeroit(q, _):
        mask_v[pl.ds(q * 16, 16)] = zero16
        return 0
    lax.fori_loop(0, EBUF // 16, zeroit, 0)

    def symscan(s0, _):
        pltpu.sync_copy(sym_hbm.at[pl.ds(s0 * SYB, SYB)], symb)

        def inner(q, _):
            v = symb[pl.ds(q * 16, 16)]
            loc = v - lo
            m = (loc >= 0) & (loc < n_loc)
            locc = jnp.clip(loc, 0, EBUF - 1)
            plsc.store_scatter(mask_v, [locc], one16, mask=m)
            return 0
        lax.fori_loop(0, SYB // 16, inner, 0)
        return 0
    lax.fori_loop(0, NSYM // SYB, symscan, 0)

    sets = ((ixr0, ixc0, it00, it10, it20, rows30, xrb0, xcb0, sg0, sw0),
            (ixr1, ixc1, it01, it11, it21, rows31, xrb1, xcb1, sg1, sw1))

    def drain_writes(s):
        _, _, _, _, _, rows3, xrb, xcb, _, sw = s
        pltpu.make_async_copy(xr_hbm.at[pl.ds(0, CH)], xrb, sw).wait()
        pltpu.make_async_copy(xr_hbm.at[pl.ds(0, CH)], xcb, sw).wait()
        pltpu.make_async_copy(ea_hbm.at[pl.ds(0, CH)], rows3.at[0], sw).wait()

    def fire(j, s):
        ixr, ixc, it0, it1, it2, rows3, xrb, xcb, sg, _ = s
        lb = j * CH
        for q in range(CH // 16):
            sl = pl.ds(lb + q * 16, 16)
            so = pl.ds(q * 16, 16)
            mv = mask_v[sl] * VOCAB
            it0[so] = tk0[sl] + mv
            it1[so] = tk1[sl] + mv
            it2[so] = tk2[sl] + mv
            ixr[so] = rowf[sl]
            ixc[so] = colf[sl]
        pltpu.async_copy(xsum_hbm.at[ixr], xrb, sg)
        pltpu.async_copy(xsum_hbm.at[ixc], xcb, sg)
        pltpu.async_copy(emb2_hbm.at[it0], rows3.at[0], sg)
        pltpu.async_copy(emb2_hbm.at[it1], rows3.at[1], sg)
        pltpu.async_copy(emb2_hbm.at[it2], rows3.at[2], sg)

    def complete(j, s):
        _, _, _, _, _, rows3, xrb, xcb, sg, sw = s
        pltpu.make_async_copy(xsum_hbm.at[pl.ds(0, CH)], xrb, sg).wait()
        pltpu.make_async_copy(xsum_hbm.at[pl.ds(0, CH)], xcb, sg).wait()
        for t in range(3):
            pltpu.make_async_copy(emb2_hbm.at[pl.ds(0, CH)], rows3.at[t], sg).wait()

        def body(r, _):
            for k in range(D // 16):
                sk = pl.ds(k * 16, 16)
                rows3[0, r, sk] = rows3[0, r, sk] + rows3[1, r, sk] + rows3[2, r, sk]
            return 0
        lax.fori_loop(0, CH, body, 0)
        g = lo + j * CH
        pltpu.async_copy(xrb, xr_hbm.at[pl.ds(g, CH)], sw)
        pltpu.async_copy(xcb, xc_hbm.at[pl.ds(g, CH)], sw)
        pltpu.async_copy(rows3.at[0], ea_hbm.at[pl.ds(g, CH)], sw)

    def pair(jj, _):
        j0 = jj * 2
        j1 = jj * 2 + 1

        @pl.when(j0 < n_ch)
        def _():
            @pl.when(j0 >= 2)
            def _():
                drain_writes(sets[0])
            fire(j0, sets[0])

        @pl.when((j0 >= 1) & (j0 - 1 < n_ch))
        def _():
            complete(j0 - 1, sets[1])

        @pl.when(j1 < n_ch)
        def _():
            @pl.when(j1 >= 2)
            def _():
                drain_writes(sets[1])
            fire(j1, sets[1])

        @pl.when(j1 - 1 < n_ch)
        def _():
            complete(j1 - 1, sets[0])
        return 0

    lax.fori_loop(0, (CPT + 2) // 2, pair, 0)

    drain_writes(sets[0])
    drain_writes(sets[1])


def _edge_sc(x_sum, emb2, eat_t, edge_index, added_sym_edge):
    vi = lambda shape: pltpu.VMEM(shape, jnp.int32)
    vf = lambda shape: pltpu.VMEM(shape, jnp.float32)
    bufset = [vi((CH,))] * 5 + [vf((3, CH, D)), vf((CH, D)), vf((CH, D))]
    k = pl.kernel(
        _edge_sc_kernel, mesh=_mesh(),
        compiler_params=pltpu.CompilerParams(needs_layout_passes=False),
        out_type=[jax.ShapeDtypeStruct((E, D), jnp.float32)] * 3,
        scratch_types=[
            vi((EBUF,)), vi((SYB,)),
            vi((EBUF,)), vi((EBUF,)), vi((EBUF,)), vi((EBUF,)), vi((EBUF,)),
        ] + bufset + bufset + [
            pltpu.SemaphoreType.DMA, pltpu.SemaphoreType.DMA,
            pltpu.SemaphoreType.DMA, pltpu.SemaphoreType.DMA,
        ])
    return k(x_sum, emb2, eat_t, edge_index, added_sym_edge)


def _scatter_sc_kernel(msg_hbm, ei_hbm, part_hbm, cntp_hbm,
                       colf, idxb, msgb, idxb1, msgb1, ones_v, zrow, zcnt,
                       acc_sh, cnt_sh, sg0, sg1, sv0, sv1):
    sid = lax.axis_index("s")
    cid = lax.axis_index("c")
    w = sid * 2 + cid
    lo = _chunk_start(w) * CH

    zero16 = jnp.zeros((16,), jnp.float32)
    one16 = jnp.full((16,), 1.0, jnp.float32)

    def zr(r, _):
        for k in range(D // 16):
            zrow[r, pl.ds(k * 16, 16)] = zero16
        return 0
    lax.fori_loop(0, ZR, zr, 0)

    def zc(r, _):
        zcnt[r, pl.ds(0, 16)] = zero16
        return 0
    lax.fori_loop(0, 312, zc, 0)

    def ov(r, _):
        ones_v[r, pl.ds(0, 16)] = one16
        return 0
    lax.fori_loop(0, CH, ov, 0)

    pltpu.sync_copy(ei_hbm.at[pl.ds(E + lo, CPT * CH)], colf.at[pl.ds(0, CPT * CH)])

    @pl.when(w < XTRA)
    def _():
        pltpu.sync_copy(ei_hbm.at[pl.ds(E + lo + CPT * CH, CH)],
                        colf.at[pl.ds(CPT * CH, CH)])

    for p in range(2):
        nb = p * PH
        pltpu.sync_copy(zrow, acc_sh.at[pl.ds(sid * 312, ZR)])
        pltpu.sync_copy(zrow, acc_sh.at[pl.ds(sid * 312 + ZR, ZR)])
        pltpu.sync_copy(zcnt, cnt_sh.at[pl.ds(sid * 312, 312)])

        @pl.when(sid == 15)
        def _():
            pltpu.sync_copy(zrow.at[pl.ds(0, PACC - 4992)],
                            acc_sh.at[pl.ds(4992, PACC - 4992)])
            pltpu.sync_copy(zcnt.at[pl.ds(0, PACC - 4992)],
                            cnt_sh.at[pl.ds(4992, PACC - 4992)])

        plsc.subcore_barrier()

        n_ch = _chunk_start(w + 1) - _chunk_start(w)
        ssets = ((idxb, msgb, sg0, sv0), (idxb1, msgb1, sg1, sv1))

        def drain_scat(s):
            idxs, msgs, _, sv = s
            pltpu.make_async_copy(msg_hbm.at[pl.ds(0, CH)], msgs, sv).wait()
            pltpu.make_async_copy(cntp_hbm.at[0, pl.ds(0, CH)], ones_v, sv).wait()

        def fire_stage(j, s):
            _, msgs, sg, _ = s
            pltpu.async_copy(msg_hbm.at[pl.ds(lo + j * CH, CH)], msgs, sg)

        def fire_scat(j, s):
            idxs, msgs, sg, sv = s
            pltpu.make_async_copy(msg_hbm.at[pl.ds(0, CH)], msgs, sg).wait()
            lb = j * CH
            for q in range(CH // 16):
                loc = colf[pl.ds(lb + q * 16, 16)] - nb
                m = (loc >= 0) & (loc < PH)
                idxs[pl.ds(q * 16, 16)] = jnp.where(m, loc, PH)
            pltpu.async_copy(msgs, acc_sh.at[idxs], sv, add=True)
            pltpu.async_copy(ones_v, cnt_sh.at[idxs], sv, add=True)

        def spair(jj, _):
            j0 = jj * 2
            j1 = jj * 2 + 1

            @pl.when(j0 < n_ch)
            def _():
                @pl.when(j0 >= 2)
                def _():
                    drain_scat(ssets[0])
                fire_stage(j0, ssets[0])

            @pl.when((j0 >= 1) & (j0 - 1 < n_ch))
            def _():
                fire_scat(j0 - 1, ssets[1])

            @pl.when(j1 < n_ch)
            def _():
                @pl.when(j1 >= 2)
                def _():
                    drain_scat(ssets[1])
                fire_stage(j1, ssets[1])

            @pl.when(j1 - 1 < n_ch)
            def _():
                fire_scat(j1 - 1, ssets[0])
            return 0

        lax.fori_loop(0, (CPT + 2) // 2, spair, 0)
        drain_scat(ssets[0])
        drain_scat(ssets[1])

        plsc.subcore_barrier()
        sl = pl.ds(sid * 312, 312)
        glb = pl.ds(nb + sid * 312, 312)
        pltpu.sync_copy(acc_sh.at[sl], part_hbm.at[cid, glb])
        pltpu.sync_copy(cnt_sh.at[sl], cntp_hbm.at[cid, glb])

        @pl.when(sid == 15)
        def _():
            sl2 = pl.ds(4992, 8)
            glb2 = pl.ds(nb + 4992, 8)
            pltpu.sync_copy(acc_sh.at[sl2], part_hbm.at[cid, glb2])
            pltpu.sync_copy(cnt_sh.at[sl2], cntp_hbm.at[cid, glb2])


def _scatter_sc(msg, edge_index_flat):
    vf = lambda shape: pltpu.VMEM(shape, jnp.float32)
    k = pl.kernel(
        _scatter_sc_kernel, mesh=_mesh(),
        compiler_params=pltpu.CompilerParams(needs_layout_passes=False,
                                             use_tc_tiling_on_sc=False),
        out_type=[jax.ShapeDtypeStruct((2, N, D), jnp.float32),
                  jax.ShapeDtypeStruct((2, N, 16), jnp.float32)],
        scratch_types=[
            pltpu.VMEM((EBUF,), jnp.int32), pltpu.VMEM((CH,), jnp.int32),
            vf((CH, D)), pltpu.VMEM((CH,), jnp.int32), vf((CH, D)),
            vf((CH, 16)), vf((ZR, D)), vf((312, 16)),
            pltpu.VMEM_SHARED((PACC, D), jnp.float32),
            pltpu.VMEM_SHARED((PACC, 16), jnp.float32),
            pltpu.SemaphoreType.DMA, pltpu.SemaphoreType.DMA,
            pltpu.SemaphoreType.DMA, pltpu.SemaphoreType.DMA,
        ])
    return k(msg, edge_index_flat)


def _edge_block_kernel(xr_ref, xc_ref, ea_ref,
                       ew1_ref, eb1_ref, ew2_ref, eb2_ref,
                       nw1_ref, nb1_ref, nw2_ref, nb2_ref,
                       eout_ref, msg_ref):
    xr = xr_ref[...]
    xc = xc_ref[...]
    ea = ea_ref[...]
    w_r = ew1_ref[0:D, :]
    w_c = ew1_ref[D:2 * D, :]
    w_a = ew1_ref[2 * D:3 * D, :]
    h = jnp.dot(xr, w_r, preferred_element_type=jnp.float32)
    h += jnp.dot(xc, w_c, preferred_element_type=jnp.float32)
    h += jnp.dot(ea, w_a, preferred_element_type=jnp.float32)
    h = jnp.maximum(h + eb1_ref[...], 0.0)
    eout = jnp.dot(h, ew2_ref[...], preferred_element_type=jnp.float32) + eb2_ref[...]
    eout_ref[...] = eout

    m_r = nw1_ref[0:D, :]
    m_e = nw1_ref[D:2 * D, :]
    m = jnp.dot(xr, m_r, preferred_element_type=jnp.float32)
    m += jnp.dot(eout, m_e, preferred_element_type=jnp.float32)
    m = jnp.maximum(m + nb1_ref[...], 0.0)
    msg_ref[...] = jnp.dot(m, nw2_ref[...], preferred_element_type=jnp.float32) + nb2_ref[...]


def _edge_mlps(xr, xc, ea, e_w1, e_b1, e_w2, e_b2, n1_w1, n1_b1, n1_w2, n1_b2):
    nblk = E // BE
    full = lambda shape: pl.BlockSpec(shape, lambda i: (0,) * len(shape))
    blk = pl.BlockSpec((BE, D), lambda i: (i, 0))
    return pl.pallas_call(
        _edge_block_kernel,
        grid=(nblk,),
        in_specs=[blk, blk, blk,
                  full((3 * D, D)), full((1, D)), full((D, D)), full((1, D)),
                  full((2 * D, D)), full((1, D)), full((D, D)), full((1, D))],
        out_specs=[blk, blk],
        out_shape=[jax.ShapeDtypeStruct((E, D), jnp.float32),
                   jax.ShapeDtypeStruct((E, D), jnp.float32)],
    )(xr, xc, ea, e_w1, e_b1, e_w2, e_b2, n1_w1, n1_b1, n1_w2, n1_b2)


def _node_block_kernel(xs_ref, part_ref, cntp_ref, batch_row_ref,
                       w1_ref, b1_ref, w2_ref, b2_ref,
                       xout_ref, stats_ref):
    i = pl.program_id(0)
    xs = xs_ref[...]
    cnt = (cntp_ref[0] + cntp_ref[1])[:, 0:1]
    agg = (part_ref[0] + part_ref[1]) / jnp.maximum(cnt, 1.0)
    w_x = w1_ref[0:D, :]
    w_a = w1_ref[D:2 * D, :]
    h = jnp.dot(xs, w_x, preferred_element_type=jnp.float32)
    h += jnp.dot(agg, w_a, preferred_element_type=jnp.float32)
    h = jnp.maximum(h + b1_ref[...], 0.0)
    xo = jnp.dot(h, w2_ref[...], preferred_element_type=jnp.float32) + b2_ref[...]
    xout_ref[...] = xo

    gids = lax.broadcasted_iota(jnp.int32, (NG, BN), 0)
    onehot_t = (gids == batch_row_ref[0]).astype(jnp.float32)
    cat = jnp.concatenate([xo, xo * xo, jnp.ones((BN, D), jnp.float32)], axis=1)
    part = jnp.dot(onehot_t, cat, preferred_element_type=jnp.float32)

    @pl.when(i == 0)
    def _():
        stats_ref[...] = jnp.zeros_like(stats_ref)

    stats_ref[...] += part


def _node_mlp_stats(x_sum, part, cntp, batch_row, n2_w1, n2_b1, n2_w2, n2_b2):
    nblk = N // BN
    full = lambda shape: pl.BlockSpec(shape, lambda i: (0,) * len(shape))
    blk = pl.BlockSpec((BN, D), lambda i: (i, 0))
    return pl.pallas_call(
        _node_block_kernel,
        grid=(nblk,),
        in_specs=[blk, pl.BlockSpec((2, BN, D), lambda i: (0, i, 0)),
                  pl.BlockSpec((2, BN, 16), lambda i: (0, i, 0)),
                  pl.BlockSpec((1, 1, BN), lambda i: (i, 0, 0)),
                  full((2 * D, D)), full((1, D)), full((D, D)), full((1, D))],
        out_specs=[blk, full((NG, 3 * D))],
        out_shape=[jax.ShapeDtypeStruct((N, D), jnp.float32),
                   jax.ShapeDtypeStruct((NG, 3 * D), jnp.float32)],
    )(x_sum, part, cntp, batch_row, n2_w1, n2_b1, n2_w2, n2_b2)


def _norm_block_kernel(xo_ref, batch_col_ref, stats_ref, gamma_ref, beta_ref, out_ref):
    xo = xo_ref[...]
    gids = lax.broadcasted_iota(jnp.int32, (BN, NG), 1)
    onehot = (gids == batch_col_ref[...]).astype(jnp.float32)
    st = jnp.dot(onehot, stats_ref[...], preferred_element_type=jnp.float32)
    cnt = jnp.maximum(st[:, 2 * D:2 * D + 1], 1.0)
    mean = st[:, 0:D] / cnt
    var = st[:, D:2 * D] / cnt - mean * mean
    inv = lax.rsqrt(jnp.maximum(var, 0.0) + 1e-5)
    out_ref[...] = (xo - mean) * inv * gamma_ref[...] + beta_ref[...]


def _graph_norm(x_out, batch_col, stats, gamma, beta):
    nblk = N // BN
    full = lambda shape: pl.BlockSpec(shape, lambda i: (0,) * len(shape))
    blk = pl.BlockSpec((BN, D), lambda i: (i, 0))
    return pl.pallas_call(
        _norm_block_kernel,
        grid=(nblk,),
        in_specs=[blk, pl.BlockSpec((BN, 1), lambda i: (i, 0)),
                  full((NG, 3 * D)), full((1, D)), full((1, D))],
        out_specs=blk,
        out_shape=jax.ShapeDtypeStruct((N, D), jnp.float32),
    )(x_out, batch_col, stats, gamma, beta)


def kernel(x_tokens, edge_index, edge_attr_tokens, added_sym_edge, batch, emb,
           e_w1, e_b1, e_w2, e_b2, n1_w1, n1_b1, n1_w2, n1_b2,
           n2_w1, n2_b1, n2_w2, n2_b2, gamma, beta):
    x_tokens = x_tokens.astype(jnp.int32)
    edge_attr_tokens = edge_attr_tokens.astype(jnp.int32)
    edge_index = edge_index.astype(jnp.int32)
    added_sym_edge = added_sym_edge.astype(jnp.int32)
    batch = batch.astype(jnp.int32)

    x_sum = _xsum_sc(emb, jnp.transpose(x_tokens).reshape(-1))

    emb2 = jnp.concatenate([emb, -emb], axis=0)
    xr, xc, ea_sum = _edge_sc(x_sum, emb2,
                              jnp.transpose(edge_attr_tokens).reshape(-1),
                              edge_index.reshape(-1), added_sym_edge)

    r2 = lambda v: v.reshape(1, D)
    edge_out, msg = _edge_mlps(xr, xc, ea_sum, e_w1, r2(e_b1), e_w2, r2(e_b2),
                               n1_w1, r2(n1_b1), n1_w2, r2(n1_b2))

    part, cntp = _scatter_sc(msg, edge_index.reshape(-1))

    batch_row = batch.reshape(N // BN, 1, BN)
    x_out, stats = _node_mlp_stats(x_sum, part, cntp, batch_row,
                                   n2_w1, r2(n2_b1), n2_w2, r2(n2_b2))

    x_norm = _graph_norm(x_out, batch.reshape(N, 1), stats, r2(gamma), r2(beta))
    return (x_norm, edge_out)
